# baseline (speedup 1.0000x reference)
import jax
import jax.numpy as jnp
from jax.experimental import pallas as pl
from jax.experimental.pallas import tpu as pltpu

_B = 8
_T_TEXT = 128
_T_FEATS = 512
_ADIM = 256
_ODIM = 80
_HID = 256
_SIGMA = 10.0


def _fused_kernel(text_len_ref, feats_len_ref,
                  xs_ref, ys_ref,
                  W1p_ref, b1p_ref, W2p_ref, b2p_ref,
                  W1q_ref, b1q_ref, W2q_ref, b2q_ref,
                  out_ref, p_ref, q_ref, func_ref):
    t_col_i = jax.lax.broadcasted_iota(jnp.int32, (_T_FEATS, 1), 0)
    t_col = t_col_i.astype(jnp.float32)
    src = jax.lax.broadcasted_iota(jnp.int32, (_T_FEATS, _T_TEXT), 1)
    s_row = jax.lax.broadcasted_iota(jnp.int32, (1, _T_FEATS), 1)

    xi_parts = []
    for b in range(_B):
        tl_i = text_len_ref[b]
        ratio = tl_i.astype(jnp.float32) / feats_len_ref[b].astype(jnp.float32)
        idx = jnp.floor(t_col * ratio).astype(jnp.int32)
        idx = jnp.minimum(idx, tl_i - 1)
        onehot = (src == idx).astype(jnp.float32)
        xs_b = xs_ref[b]
        xs_hi = xs_b.astype(jnp.bfloat16).astype(jnp.float32)
        xs_lo = xs_b - xs_hi
        xi_parts.append(
            jnp.dot(onehot, xs_hi, preferred_element_type=jnp.float32)
            + jnp.dot(onehot, xs_lo, preferred_element_type=jnp.float32))
    Xi = jnp.concatenate(xi_parts, axis=0)

    H_p = jnp.tanh(jnp.dot(Xi, W1p_ref[:],
                           preferred_element_type=jnp.float32) + b1p_ref[:])
    out_p = jnp.dot(H_p, W2p_ref[:],
                    preferred_element_type=jnp.float32) + b2p_ref[:]
    p_ref[...] = out_p.reshape(_B, _T_FEATS, 2)

    Ys = ys_ref[...].reshape(_B * _T_FEATS, _ODIM)
    H_q = jnp.tanh(jnp.dot(Xi, W1q_ref[:_ADIM],
                           preferred_element_type=jnp.float32)
                   + jnp.dot(Ys, W1q_ref[_ADIM:],
                             preferred_element_type=jnp.float32)
                   + b1q_ref[:])
    out_q = jnp.dot(H_q, W2q_ref[:],
                    preferred_element_type=jnp.float32) + b2q_ref[:]
    q_ref[...] = out_q.reshape(_B, _T_FEATS, 2)

    z_cols = []
    valids = []
    for b in range(_B):
        fl_i = feats_len_ref[b]
        valid = t_col_i < fl_i
        valids.append(valid)
        r0 = b * _T_FEATS
        mu2 = jnp.concatenate([out_p[r0:r0 + _T_FEATS, 0:1],
                               out_q[r0:r0 + _T_FEATS, 0:1]], axis=1)
        z2 = jnp.concatenate([jnp.zeros((1, 2), jnp.float32), mu2[:-1]], axis=0)
        z2 = jnp.where(valid, z2, 0.0)
        z2 = z2 - jnp.sum(z2, axis=0, keepdims=True) / fl_i.astype(jnp.float32)
        z_cols.append(z2)
    Z = jnp.concatenate(z_cols, axis=1)
    ti = jax.lax.broadcasted_iota(jnp.int32, (_T_FEATS, _T_FEATS), 0)
    si = jax.lax.broadcasted_iota(jnp.int32, (_T_FEATS, _T_FEATS), 1)
    ltri = (si <= ti).astype(jnp.float32)
    CS = jnp.dot(ltri, Z, preferred_element_type=jnp.float32)

    inv = jnp.float32(1.0 / (_SIGMA * (2.0 ** 0.5)))
    total_num = jnp.float32(0.0)
    total_den = jnp.float32(0.0)
    for b in range(_B):
        fl_i = feats_len_ref[b]
        fl_f = fl_i.astype(jnp.float32)
        valid = valids[b]
        cs2 = jnp.where(valid, CS[:, 2 * b:2 * b + 2], 0.0)
        pz = cs2[:, 0:1]
        qz = cs2[:, 1:2]

        d = qz - pz
        total_num += jnp.sum(d * d * valid.astype(jnp.float32))
        total_den += fl_f

        center = t_col + qz
        s_star = jnp.clip(jnp.floor(center + 0.5), 0.0, fl_f - 1.0)
        em_col = jnp.square((center - s_star) * inv)
        cc = center * inv
        srow_f = s_row.astype(jnp.float32) * inv
        ds = cc - srow_f
        arg = jnp.minimum(em_col - ds * ds, 0.0)
        ew = jnp.exp(arg)
        valid_f = valid.astype(jnp.float32)
        r0 = b * _T_FEATS
        Xi_m = Xi[r0:r0 + _T_FEATS] * valid_f
        denom = jnp.dot(ew, valid_f, preferred_element_type=jnp.float32)
        out = jnp.dot(ew, Xi_m, preferred_element_type=jnp.float32)
        out_ref[b] = out * (valid_f / denom)

    func_ref[...] = jnp.full((1, 128), total_num / total_den, jnp.float32)


def kernel(xs, ys, text_lengths, feats_lengths,
           W1p, b1p, W2p, b2p, W1q, b1q, W2q, b2q):
    b1p2 = b1p.reshape(1, _HID)
    b1q2 = b1q.reshape(1, _HID)
    b2p2 = b2p.reshape(1, 2)
    b2q2 = b2q.reshape(1, 2)

    smem = pl.BlockSpec(memory_space=pltpu.SMEM)
    out_shapes = [
        jax.ShapeDtypeStruct((_B, _T_FEATS, _ADIM), jnp.float32),
        jax.ShapeDtypeStruct((_B, _T_FEATS, 2), jnp.float32),
        jax.ShapeDtypeStruct((_B, _T_FEATS, 2), jnp.float32),
        jax.ShapeDtypeStruct((1, 128), jnp.float32),
    ]
    xs_out, p, q, func = pl.pallas_call(
        _fused_kernel,
        in_specs=[smem, smem] + [pl.BlockSpec()] * 10,
        out_specs=[pl.BlockSpec()] * 4,
        out_shape=out_shapes,
    )(text_lengths, feats_lengths,
      xs, ys, W1p, b1p2, W2p, b2p2, W1q, b1q2, W2q, b2q2)

    return (xs_out, func[0, 0], p, q)
